# baseline (speedup 1.0000x reference)
import numpy as np
import jax
import jax.numpy as jnp
from jax.experimental import pallas as pl
from jax.experimental.pallas import tpu as pltpu

_SR = 9
_RR = 0.5
_MAXIT = 3
_D = 2 * _SR + 1
_PI = 3.141592653589793
_SSIGMA = float(np.sqrt(2.0 * _SR ** 2) / 1.5)
_RCONST = float(1.0 / (_RR * np.sqrt(2.0 * _PI)))
_RNG_THR = float(np.exp(-0.5) / (_RR * np.sqrt(2.0 * _PI)))
_H = 224
_W = 224
_SH = 32
_NS = _H // _SH
_PR = 256
_PC = 256


def _spt_table() -> np.ndarray:
    ax = np.arange(-_SR, _SR + 1, dtype=np.float32)
    dy, dx = np.meshgrid(ax, ax, indexing='ij')
    dist = np.sqrt(dy ** 2 + dx ** 2)
    w = np.exp(-0.5 * (dist / _SSIGMA) ** 2) / (_SSIGMA * np.sqrt(2.0 * _PI))
    return w.astype(np.float32)


def _ms_kernel(spt_ref, img_ref, out_ref, pad_ref, pln_ref):
    pad_ref[...] = jnp.zeros((_PR, _PC), jnp.float32)
    pad_ref[16:16 + _H, 9:9 + _W] = img_ref[0]

    def ms_iter(_, __):
        for s in range(_NS):
            for dy in range(_D):
                r0 = 7 + dy + _SH * s
                pln_ref[dy, s] = pad_ref[r0:r0 + _SH, :]

        def strip_body(s, __):
            c = pln_ref[9, s][:, 9:9 + _W]

            def dy_body(dy, nd):
                num, den = nd
                p = pln_ref[dy, s]
                for dx in range(_D):
                    nb = p[:, dx:dx + _W]
                    diff = nb - c
                    rw = jnp.exp(-2.0 * diff * diff) * _RCONST
                    rw = jnp.where(rw < _RNG_THR, 0.0, rw)
                    w = rw * spt_ref[dy, dx]
                    num = num + w * nb
                    den = den + w
                return num, den

            num, den = jax.lax.fori_loop(
                0, _D, dy_body,
                (jnp.zeros((_SH, _W), jnp.float32),
                 jnp.zeros((_SH, _W), jnp.float32)))
            pad_ref[pl.ds(16 + _SH * s, _SH), 9:9 + _W] = num / (den + 1e-8)
            return 0

        jax.lax.fori_loop(0, _NS, strip_body, 0)
        return 0

    jax.lax.fori_loop(0, _MAXIT, ms_iter, 0)
    out_ref[0] = pad_ref[16:16 + _H, 9:9 + _W]


def kernel(img):
    bs = img.shape[0]
    x = img.reshape(bs, _H, _W)
    spt = jnp.asarray(_spt_table())
    out = pl.pallas_call(
        _ms_kernel,
        grid=(bs,),
        in_specs=[
            pl.BlockSpec(memory_space=pltpu.SMEM),
            pl.BlockSpec((1, _H, _W), lambda b: (b, 0, 0)),
        ],
        out_specs=pl.BlockSpec((1, _H, _W), lambda b: (b, 0, 0)),
        out_shape=jax.ShapeDtypeStruct((bs, _H, _W), jnp.float32),
        scratch_shapes=[
            pltpu.VMEM((_PR, _PC), jnp.float32),
            pltpu.VMEM((_D, _NS, _SH, _PC), jnp.float32),
        ],
        compiler_params=pltpu.CompilerParams(
            dimension_semantics=("parallel",)),
    )(spt, x)
    return out.reshape(img.shape)


# speedup vs baseline: 1.4952x; 1.4952x over previous
import numpy as np
import jax
import jax.numpy as jnp
from jax.experimental import pallas as pl
from jax.experimental.pallas import tpu as pltpu

_SR = 9
_RR = 0.5
_MAXIT = 3
_D = 2 * _SR + 1
_PI = 3.141592653589793
_SSIGMA = float(np.sqrt(2.0 * _SR ** 2) / 1.5)
_RCONST = float(1.0 / (_RR * np.sqrt(2.0 * _PI)))
_RNG_THR = float(np.exp(-0.5) / (_RR * np.sqrt(2.0 * _PI)))
_H = 224
_W = 224
_SH = 8
_NS = _H // _SH
_PR = 256
_PC = 256


def _spt_table() -> np.ndarray:
    ax = np.arange(-_SR, _SR + 1, dtype=np.float32)
    dy, dx = np.meshgrid(ax, ax, indexing='ij')
    dist = np.sqrt(dy ** 2 + dx ** 2)
    w = np.exp(-0.5 * (dist / _SSIGMA) ** 2) / (_SSIGMA * np.sqrt(2.0 * _PI))
    return w.astype(np.float32)


def _ms_kernel(spt_ref, img_ref, out_ref, pad_ref, pln_ref):
    pad_ref[...] = jnp.zeros((_PR, _PC), jnp.float32)
    pad_ref[16:16 + _H, 9:9 + _W] = img_ref[0]

    def ms_iter(_, __):
        for s in range(_NS):
            for dy in range(_D):
                r0 = 7 + dy + _SH * s
                pln_ref[dy, s] = pad_ref[r0:r0 + _SH, :]

        def strip_body(s, __):
            c = pln_ref[9, s][:, 9:9 + _W]

            def dy_body(dy, nd):
                num, den = nd
                p = pln_ref[dy, s]
                for dx in range(_D):
                    nb = p[:, dx:dx + _W]
                    diff = nb - c
                    rw = jnp.exp(-2.0 * diff * diff) * _RCONST
                    rw = jnp.where(rw < _RNG_THR, 0.0, rw)
                    w = rw * spt_ref[dy, dx]
                    num = num + w * nb
                    den = den + w
                return num, den

            num, den = jax.lax.fori_loop(
                0, _D, dy_body,
                (jnp.zeros((_SH, _W), jnp.float32),
                 jnp.zeros((_SH, _W), jnp.float32)))
            pad_ref[pl.ds(16 + _SH * s, _SH), 9:9 + _W] = num / (den + 1e-8)
            return 0

        jax.lax.fori_loop(0, _NS, strip_body, 0)
        return 0

    jax.lax.fori_loop(0, _MAXIT, ms_iter, 0)
    out_ref[0] = pad_ref[16:16 + _H, 9:9 + _W]


def kernel(img):
    bs = img.shape[0]
    x = img.reshape(bs, _H, _W)
    spt = jnp.asarray(_spt_table())
    out = pl.pallas_call(
        _ms_kernel,
        grid=(bs,),
        in_specs=[
            pl.BlockSpec(memory_space=pltpu.SMEM),
            pl.BlockSpec((1, _H, _W), lambda b: (b, 0, 0)),
        ],
        out_specs=pl.BlockSpec((1, _H, _W), lambda b: (b, 0, 0)),
        out_shape=jax.ShapeDtypeStruct((bs, _H, _W), jnp.float32),
        scratch_shapes=[
            pltpu.VMEM((_PR, _PC), jnp.float32),
            pltpu.VMEM((_D, _NS, _SH, _PC), jnp.float32),
        ],
        compiler_params=pltpu.CompilerParams(
            dimension_semantics=("parallel",)),
    )(spt, x)
    return out.reshape(img.shape)


# speedup vs baseline: 1.5117x; 1.0111x over previous
import numpy as np
import jax
import jax.numpy as jnp
from jax.experimental import pallas as pl
from jax.experimental.pallas import tpu as pltpu

_SR = 9
_RR = 0.5
_MAXIT = 3
_D = 2 * _SR + 1
_PI = 3.141592653589793
_SSIGMA = float(np.sqrt(2.0 * _SR ** 2) / 1.5)
_RCONST = float(1.0 / (_RR * np.sqrt(2.0 * _PI)))
_RNG_THR = float(np.exp(-0.5) / (_RR * np.sqrt(2.0 * _PI)))
_N2L2E = float(-2.0 * np.log2(np.e))
_R2 = _RR * _RR
_H = 224
_W = 224
_SH = 8
_NS = _H // _SH
_PR = 256
_PC = 256


def _spt_table() -> np.ndarray:
    ax = np.arange(-_SR, _SR + 1, dtype=np.float32)
    dy, dx = np.meshgrid(ax, ax, indexing='ij')
    dist = np.sqrt(dy ** 2 + dx ** 2)
    w = np.exp(-0.5 * (dist / _SSIGMA) ** 2) / (_SSIGMA * np.sqrt(2.0 * _PI))
    return (w * _RCONST).astype(np.float32)


def _ms_kernel(spt_ref, img_ref, out_ref, pad_ref, pln_ref):
    pad_ref[...] = jnp.zeros((_PR, _PC), jnp.float32)
    pad_ref[16:16 + _H, 9:9 + _W] = img_ref[0]

    def ms_iter(_, __):
        for s in range(_NS):
            for dy in range(_D):
                r0 = 7 + dy + _SH * s
                pln_ref[dy, s] = pad_ref[r0:r0 + _SH, :]

        def strip_body(s, __):
            c = pln_ref[9, s][:, 9:9 + _W]

            def dy_body(dy, nd):
                num, den = nd
                p = pln_ref[dy, s]
                for dx in range(_D):
                    nb = p[:, dx:dx + _W]
                    diff = nb - c
                    d2 = diff * diff
                    e = jnp.exp2(d2 * _N2L2E)
                    w = jnp.where(d2 > _R2, 0.0, e) * spt_ref[dy, dx]
                    num = num + w * nb
                    den = den + w
                return num, den

            num, den = jax.lax.fori_loop(
                0, _D, dy_body,
                (jnp.zeros((_SH, _W), jnp.float32),
                 jnp.zeros((_SH, _W), jnp.float32)))
            pad_ref[pl.ds(16 + _SH * s, _SH), 9:9 + _W] = num / (den + 1e-8)
            return 0

        jax.lax.fori_loop(0, _NS, strip_body, 0)
        return 0

    jax.lax.fori_loop(0, _MAXIT, ms_iter, 0)
    out_ref[0] = pad_ref[16:16 + _H, 9:9 + _W]


def kernel(img):
    bs = img.shape[0]
    x = img.reshape(bs, _H, _W)
    spt = jnp.asarray(_spt_table())
    out = pl.pallas_call(
        _ms_kernel,
        grid=(bs,),
        in_specs=[
            pl.BlockSpec(memory_space=pltpu.SMEM),
            pl.BlockSpec((1, _H, _W), lambda b: (b, 0, 0)),
        ],
        out_specs=pl.BlockSpec((1, _H, _W), lambda b: (b, 0, 0)),
        out_shape=jax.ShapeDtypeStruct((bs, _H, _W), jnp.float32),
        scratch_shapes=[
            pltpu.VMEM((_PR, _PC), jnp.float32),
            pltpu.VMEM((_D, _NS, _SH, _PC), jnp.float32),
        ],
        compiler_params=pltpu.CompilerParams(
            dimension_semantics=("parallel",)),
    )(spt, x)
    return out.reshape(img.shape)
